# speedup vs baseline: 2.1165x; 2.1165x over previous
import functools

import jax
import jax.numpy as jnp
from jax import lax
from jax.experimental import pallas as pl
from jax.experimental.pallas import tpu as pltpu
from jax.experimental.pallas import tpu_sc as plsc

F32 = jnp.float32
NC = 2
NS = 16
NW = NC * NS


def _pick_block(n):
    if n <= 4096:
        return n
    for b in (2048, 2000, 1024, 1000, 512, 500, 256, 128, 64, 8):
        if n % b == 0:
            return b
    return n


def _fused_mlp(terms, extras, residual, p, interpret=False):
    n = terms[0][0].shape[0]
    d = p["w2"].shape[1]
    blk = _pick_block(n)
    grid = n // blk
    nt = len(terms)
    ne = len(extras)
    has_res = residual is not None

    def body(*refs):
        xs = refs[:nt]
        ws = refs[nt:2 * nt]
        exs = refs[2 * nt:2 * nt + ne]
        pos = 2 * nt + ne
        res = refs[pos] if has_res else None
        pos += 1 if has_res else 0
        b1r, w2r, b2r, gr, br = refs[pos:pos + 5]
        outr = refs[pos + 5]
        s = jnp.dot(xs[0][...], ws[0][...], preferred_element_type=F32)
        for i in range(1, nt):
            s = s + jnp.dot(xs[i][...], ws[i][...], preferred_element_type=F32)
        s = s + b1r[...]
        for ex in exs:
            s = s + ex[...]
        h = s * jax.nn.sigmoid(s)
        y = jnp.dot(h, w2r[...], preferred_element_type=F32) + b2r[...]
        mu = jnp.mean(y, axis=-1, keepdims=True)
        var = jnp.mean((y - mu) * (y - mu), axis=-1, keepdims=True)
        o = (y - mu) * lax.rsqrt(var + 1e-5) * gr[...] + br[...]
        if has_res:
            o = o + res[...]
        outr[...] = o

    in_specs = []
    args = []
    for x, _ in terms:
        in_specs.append(pl.BlockSpec((blk, x.shape[1]), lambda i: (i, 0)))
        args.append(x)
    for _, w in terms:
        in_specs.append(pl.BlockSpec(w.shape, lambda i: (0, 0)))
        args.append(w)
    for ex in extras:
        in_specs.append(pl.BlockSpec((blk, d), lambda i: (i, 0)))
        args.append(ex)
    if has_res:
        in_specs.append(pl.BlockSpec((blk, d), lambda i: (i, 0)))
        args.append(residual)
    vecs = [p["b1"].reshape(1, -1), p["w2"], p["b2"].reshape(1, -1),
            p["g"].reshape(1, -1), p["b"].reshape(1, -1)]
    for v in vecs:
        in_specs.append(pl.BlockSpec(v.shape, lambda i: (0, 0)))
        args.append(v)

    return pl.pallas_call(
        body,
        grid=(grid,),
        in_specs=in_specs,
        out_specs=pl.BlockSpec((blk, d), lambda i: (i, 0)),
        out_shape=jax.ShapeDtypeStruct((n, d), F32),
        interpret=interpret,
    )(*args)


def _matmul(x, w, interpret=False):
    n, k = x.shape
    d = w.shape[1]
    blk = _pick_block(n)

    def body(xr, wr, outr):
        outr[...] = jnp.dot(xr[...], wr[...], preferred_element_type=F32)

    return pl.pallas_call(
        body,
        grid=(n // blk,),
        in_specs=[pl.BlockSpec((blk, k), lambda i: (i, 0)),
                  pl.BlockSpec((k, d), lambda i: (0, 0))],
        out_specs=pl.BlockSpec((blk, d), lambda i: (i, 0)),
        out_shape=jax.ShapeDtypeStruct((n, d), F32),
        interpret=interpret,
    )(x, w)


def _chunk(n):
    for c in (128, 120, 112, 104, 96, 88, 80, 72, 64, 56, 48, 40, 32, 24, 16, 8):
        if n % c == 0:
            return c
    raise ValueError(f"no chunk for {n}")


def _sc_gather2(ts, td, src, dst):
    e, d = src.shape[0], ts.shape[1]
    n = e // NW
    c = _chunk(n)
    nch = n // c
    mesh = plsc.VectorSubcoreMesh(core_axis_name="c", subcore_axis_name="s")

    @functools.partial(
        pl.kernel, mesh=mesh,
        out_type=(jax.ShapeDtypeStruct((e, d), F32),
                  jax.ShapeDtypeStruct((e, d), F32)),
        scratch_types=[
            pltpu.VMEM((c,), jnp.int32), pltpu.VMEM((c,), jnp.int32),
            pltpu.VMEM((c, d), F32), pltpu.VMEM((c, d), F32),
            pltpu.SemaphoreType.DMA, pltpu.SemaphoreType.DMA,
        ],
    )
    def k(ts_h, td_h, src_h, dst_h, os_h, od_h, is_v, id_v, rs_v, rd_v, sa, sb):
        wid = lax.axis_index("s") * NC + lax.axis_index("c")
        base = wid * n

        def bodyf(j, carry):
            off = base + j * c
            pltpu.sync_copy(src_h.at[pl.ds(off, c)], is_v)
            pltpu.sync_copy(dst_h.at[pl.ds(off, c)], id_v)
            ca = pltpu.async_copy(ts_h.at[is_v], rs_v, sa)
            cb = pltpu.async_copy(td_h.at[id_v], rd_v, sb)
            ca.wait()
            cb.wait()
            pltpu.sync_copy(rs_v, os_h.at[pl.ds(off, c)])
            pltpu.sync_copy(rd_v, od_h.at[pl.ds(off, c)])
            return carry

        lax.fori_loop(0, nch, bodyf, 0)

    return k(ts, td, src, dst)


def _sc_scatter_add(y, dst, r):
    e, d = y.shape
    n = e // NW
    c = _chunk(n)
    nch = n // c
    zeros = jnp.zeros((r, d), F32)
    mesh = plsc.VectorSubcoreMesh(core_axis_name="c", subcore_axis_name="s")

    @functools.partial(
        pl.kernel, mesh=mesh,
        out_type=jax.ShapeDtypeStruct((NC, r, d), F32),
        scratch_types=[
            pltpu.VMEM((c,), jnp.int32), pltpu.VMEM((c, d), F32),
            pltpu.VMEM_SHARED((r, d), F32),
        ],
    )
    def k(y_h, dst_h, z_h, out_h, idx_v, rows_v, shared):
        cid = lax.axis_index("c")
        sid = lax.axis_index("s")
        wid = sid * NC + cid

        @pl.when(sid == 0)
        def _():
            pltpu.sync_copy(z_h, shared)

        plsc.subcore_barrier()

        def bodyf(j, carry):
            off = wid * n + j * c
            pltpu.sync_copy(dst_h.at[pl.ds(off, c)], idx_v)
            pltpu.sync_copy(y_h.at[pl.ds(off, c)], rows_v)
            pltpu.sync_copy(rows_v, shared.at[idx_v], add=True)
            return carry

        lax.fori_loop(0, nch, bodyf, 0)
        plsc.subcore_barrier()

        @pl.when(sid == 0)
        def _():
            pltpu.sync_copy(shared, out_h.at[cid])

    return k(y, dst, zeros)


def _pad_rows(x, n):
    if x.shape[0] == n:
        return x
    return jnp.pad(x, ((0, n - x.shape[0]), (0, 0)))


def _pad_idx(ix, n, fill):
    if ix.shape[0] == n:
        return ix
    return jnp.pad(ix, (0, n - ix.shape[0]), constant_values=fill)


def _split_w1(w1, k0, k1):
    return w1[:k0], w1[k0:k0 + k1], w1[k0 + k1:]


def kernel(grid_nfeat, mesh_nfeat, g2m_efeat, mesh_efeat, m2g_efeat,
           g2m_src, g2m_dst, mesh_src, mesh_dst, m2g_src, m2g_dst, params):
    ng = grid_nfeat.shape[0]
    nm = mesh_nfeat.shape[0]
    nmp = ((nm + 7) // 8) * 8
    rm = nmp + 8
    rg = ((ng + 7) // 8) * 8 + 8
    tg_trash = rg - 8

    def pad_e(n_edges):
        return ((n_edges + 8 * NW - 1) // (8 * NW)) * (8 * NW)

    eg2m = pad_e(g2m_efeat.shape[0])
    em_n = pad_e(mesh_efeat.shape[0])
    em2g = pad_e(m2g_efeat.shape[0])

    g = _fused_mlp([(grid_nfeat, params["grid_embed"]["w1"])], [], None,
                   params["grid_embed"])
    m = _fused_mlp([(_pad_rows(mesh_nfeat, nmp), params["mesh_embed"]["w1"])],
                   [], None, params["mesh_embed"])

    ef = jnp.pad(_pad_rows(g2m_efeat, eg2m), ((0, 0), (0, 4)))
    w1e = jnp.pad(params["g2m_eembed"]["w1"], ((0, 4), (0, 0)))
    e = _fused_mlp([(ef, w1e)], [], None, params["g2m_eembed"])

    wa, wb, wc = _split_w1(params["g2m_edge"]["w1"], 128, 128)
    tsrc = _matmul(g, wb)
    tdst = _matmul(m, wc)
    qs, qd = _sc_gather2(tsrc, tdst,
                         _pad_idx(g2m_src, eg2m, 0), _pad_idx(g2m_dst, eg2m, 0))
    e = _fused_mlp([(e, wa)], [qs, qd], e, params["g2m_edge"])

    scat = _sc_scatter_add(e, _pad_idx(g2m_dst, eg2m, nmp), rm)
    wna, wnb, _ = _split_w1(params["g2m_node"]["w1"], 128, 128)
    m = _fused_mlp([(scat[0, :nmp], wna), (scat[1, :nmp], wna), (m, wnb)],
                   [], m, params["g2m_node"])
    g = _fused_mlp([(g, params["grid_enc"]["w1"])], [], g, params["grid_enc"])

    efm = jnp.pad(_pad_rows(mesh_efeat, em_n), ((0, 0), (0, 4)))
    w1m = jnp.pad(params["mesh_eembed"]["w1"], ((0, 4), (0, 0)))
    em = _fused_mlp([(efm, w1m)], [], None, params["mesh_eembed"])

    msrc = _pad_idx(mesh_src, em_n, 0)
    mdst = _pad_idx(mesh_dst, em_n, 0)
    mdst_sc = _pad_idx(mesh_dst, em_n, nmp)
    for lp in params["proc"]:
        wa, wb, wc = _split_w1(lp["edge"]["w1"], 128, 128)
        tcomb = _matmul(m, jnp.concatenate([wb, wc], axis=1))
        qs, qd = _sc_gather2(tcomb[:, :128], tcomb[:, 128:], msrc, mdst)
        em = _fused_mlp([(em, wa)], [qs, qd], em, lp["edge"])
        scat = _sc_scatter_add(em, mdst_sc, rm)
        wna, wnb, _ = _split_w1(lp["node"]["w1"], 128, 128)
        m = _fused_mlp([(scat[0, :nmp], wna), (scat[1, :nmp], wna), (m, wnb)],
                       [], m, lp["node"])

    efd = jnp.pad(_pad_rows(m2g_efeat, em2g), ((0, 0), (0, 4)))
    w1d = jnp.pad(params["m2g_eembed"]["w1"], ((0, 4), (0, 0)))
    ed = _fused_mlp([(efd, w1d)], [], None, params["m2g_eembed"])

    wa, wb, wc = _split_w1(params["m2g_edge"]["w1"], 128, 128)
    tsrc = _matmul(m, wb)
    tdst = _matmul(g, wc)
    qs, qd = _sc_gather2(tsrc, tdst,
                         _pad_idx(m2g_src, em2g, 0), _pad_idx(m2g_dst, em2g, 0))
    ed = _fused_mlp([(ed, wa)], [qs, qd], ed, params["m2g_edge"])

    scat = _sc_scatter_add(ed, _pad_idx(m2g_dst, em2g, tg_trash), rg)
    wna, wnb, _ = _split_w1(params["m2g_node"]["w1"], 128, 128)
    g = _fused_mlp([(scat[0, :ng], wna), (scat[1, :ng], wna), (g, wnb)],
                   [], g, params["m2g_node"])

    return _fused_mlp([(g, params["final"]["w1"])], [], None, params["final"])
